# baseline (speedup 1.0000x reference)
import functools

import jax
import jax.numpy as jnp
from jax import lax
from jax.experimental import pallas as pl
from jax.experimental.pallas import tpu as pltpu
from jax.experimental.pallas import tpu_sc as plsc

HEIGHT, WIDTH, DIM, BATCH = 32, 32, 768, 64
LANES = 16
NC, NS = 2, 16

TC_BATCH = 40
SC_BATCH = BATCH - TC_BATCH

_mesh = plsc.VectorSubcoreMesh(core_axis_name="c", subcore_axis_name="s")


@functools.partial(
    pl.kernel,
    mesh=_mesh,
    out_type=jax.ShapeDtypeStruct((SC_BATCH, HEIGHT * WIDTH, DIM), jnp.float32),
    scratch_types=[
        pltpu.VMEM((WIDTH, DIM), jnp.float32),
        pltpu.VMEM((DIM,), jnp.float32),
        pltpu.SemaphoreType.DMA,
    ],
)
def _sc_pos_kernel(row_hbm, col_hbm, out_hbm, buf_v, row_v, sem):
    wid = lax.axis_index("s") * NC + lax.axis_index("c")
    pltpu.sync_copy(col_hbm, buf_v)
    pltpu.sync_copy(row_hbm.at[wid], row_v)

    def add_row(w, carry):
        for j in range(DIM // LANES):
            sl = pl.ds(j * LANES, LANES)
            buf_v[w, sl] = buf_v[w, sl] + row_v[sl]
        return carry

    lax.fori_loop(0, WIDTH, add_row, 0)

    base = wid * WIDTH
    group = 8
    prev = None
    for g in range((SC_BATCH + group - 1) // group):
        cur = [
            pltpu.async_copy(buf_v, out_hbm.at[b, pl.ds(base, WIDTH)], sem)
            for b in range(g * group, min((g + 1) * group, SC_BATCH))
        ]
        if prev is not None:
            for c in prev:
                c.wait()
        prev = cur
    for c in prev:
        c.wait()


def _tc_pos_kernel(row_ref, col_ref, out_ref):
    r = row_ref[:]
    c = col_ref[:]
    pos = (r[:, None, :] + c[None, :, :]).reshape(HEIGHT * WIDTH, DIM)
    out_ref[0] = pos


def kernel(batch_size, row_embed, col_embed):
    del batch_size
    tc_part = pl.pallas_call(
        _tc_pos_kernel,
        grid=(TC_BATCH,),
        in_specs=[
            pl.BlockSpec((HEIGHT, DIM), lambda b: (0, 0)),
            pl.BlockSpec((WIDTH, DIM), lambda b: (0, 0)),
        ],
        out_specs=pl.BlockSpec((1, HEIGHT * WIDTH, DIM), lambda b: (b, 0, 0)),
        out_shape=jax.ShapeDtypeStruct((TC_BATCH, HEIGHT * WIDTH, DIM), jnp.float32),
    )(row_embed, col_embed)
    sc_part = _sc_pos_kernel(row_embed, col_embed)
    return lax.concatenate([tc_part, sc_part], 0)


# speedup vs baseline: 2.2346x; 2.2346x over previous
import functools

import jax
import jax.numpy as jnp
from jax import lax
from jax.experimental import pallas as pl
from jax.experimental.pallas import tpu as pltpu
from jax.experimental.pallas import tpu_sc as plsc

HEIGHT, WIDTH, DIM, BATCH = 32, 32, 768, 64
LANES = 16
NC, NS = 2, 16
REP = 4

_mesh = plsc.VectorSubcoreMesh(core_axis_name="c", subcore_axis_name="s")


@functools.partial(
    pl.kernel,
    mesh=_mesh,
    out_type=jax.ShapeDtypeStruct((BATCH, HEIGHT * WIDTH, DIM), jnp.float32),
    scratch_types=[
        pltpu.VMEM((REP, WIDTH, DIM), jnp.float32),
        pltpu.VMEM((DIM,), jnp.float32),
        pltpu.SemaphoreType.DMA,
    ],
)
def _sc_pos_kernel(row_hbm, col_hbm, out_hbm, buf_v, row_v, sem):
    wid = lax.axis_index("s") * NC + lax.axis_index("c")
    ccol = pltpu.async_copy(col_hbm, buf_v.at[0], sem)
    crow = pltpu.async_copy(row_hbm.at[wid], row_v, sem)
    ccol.wait()
    crow.wait()

    def add_row(w, carry):
        for j in range(DIM // LANES):
            sl = pl.ds(j * LANES, LANES)
            v = buf_v[0, w, sl] + row_v[sl]
            for r in range(REP):
                buf_v[r, w, sl] = v
        return carry

    lax.fori_loop(0, WIDTH, add_row, 0)

    base = wid * WIDTH
    copies = [
        pltpu.async_copy(
            buf_v, out_hbm.at[pl.ds(b, REP), pl.ds(base, WIDTH)], sem
        )
        for b in range(0, BATCH, REP)
    ]
    for c in copies:
        c.wait()


def kernel(batch_size, row_embed, col_embed):
    del batch_size
    return _sc_pos_kernel(row_embed, col_embed)


# speedup vs baseline: 2.2868x; 1.0234x over previous
import functools

import jax
import jax.numpy as jnp
from jax import lax
from jax.experimental import pallas as pl
from jax.experimental.pallas import tpu as pltpu
from jax.experimental.pallas import tpu_sc as plsc

HEIGHT, WIDTH, DIM, BATCH = 32, 32, 768, 64
LANES = 16
NC, NS = 2, 16
HALF = WIDTH // 2

_mesh = plsc.VectorSubcoreMesh(core_axis_name="c", subcore_axis_name="s")


@functools.partial(
    pl.kernel,
    mesh=_mesh,
    out_type=jax.ShapeDtypeStruct((BATCH, HEIGHT * WIDTH, DIM), jnp.float32),
    scratch_types=[
        pltpu.VMEM((WIDTH, DIM), jnp.float32),
        pltpu.VMEM((DIM,), jnp.float32),
        pltpu.SemaphoreType.DMA,
    ],
)
def _sc_pos_kernel(row_hbm, col_hbm, out_hbm, buf_v, row_v, sem):
    wid = lax.axis_index("s") * NC + lax.axis_index("c")
    ccol = pltpu.async_copy(col_hbm, buf_v, sem)
    crow = pltpu.async_copy(row_hbm.at[wid], row_v, sem)
    ccol.wait()
    crow.wait()

    def add_row(w, carry):
        for j in range(DIM // LANES):
            sl = pl.ds(j * LANES, LANES)
            buf_v[w, sl] = buf_v[w, sl] + row_v[sl]
        return carry

    base = wid * WIDTH
    group = 16
    pending = []

    def stream_half(lo):
        for g in range(BATCH // group):
            cur = [
                pltpu.async_copy(
                    buf_v.at[pl.ds(lo, HALF)],
                    out_hbm.at[b, pl.ds(base + lo, HALF)],
                    sem,
                )
                for b in range(g * group, (g + 1) * group)
            ]
            if pending:
                for c in pending.pop():
                    c.wait()
            pending.append(cur)

    lax.fori_loop(0, HALF, add_row, 0)
    stream_half(0)
    lax.fori_loop(HALF, WIDTH, add_row, 0)
    stream_half(HALF)
    while pending:
        for c in pending.pop():
            c.wait()


def kernel(batch_size, row_embed, col_embed):
    del batch_size
    return _sc_pos_kernel(row_embed, col_embed)
